# Initial kernel scaffold; baseline (speedup 1.0000x reference)
import jax
import jax.numpy as jnp
from jax.experimental import pallas as pl


def kernel(inputs, pos_table):
    raise NotImplementedError("write your pallas kernel here")


# speedup vs baseline: 3.1544x; 3.1544x over previous
import jax
import jax.numpy as jnp
from jax.experimental import pallas as pl

_N_REP = 16


def _repeat_body(tbl_ref, out_ref):
    x = tbl_ref[...]
    d = x.shape[1]
    k = jax.lax.broadcasted_iota(jnp.int32, (d, d * _N_REP), 0)
    c = jax.lax.broadcasted_iota(jnp.int32, (d, d * _N_REP), 1) // _N_REP
    m = (k == c).astype(x.dtype)
    out_ref[...] = jnp.dot(x, m, preferred_element_type=jnp.float32)


def kernel(inputs, pos_table):
    seq = inputs.shape[-2]
    d = pos_table.shape[-1]
    block_rows = 512
    grid = seq // block_rows
    return pl.pallas_call(
        _repeat_body,
        grid=(grid,),
        in_specs=[pl.BlockSpec((block_rows, d), lambda i: (i, 0))],
        out_specs=pl.BlockSpec((block_rows, d * _N_REP), lambda i: (i, 0)),
        out_shape=jax.ShapeDtypeStruct((seq, d * _N_REP), jnp.float32),
    )(pos_table[:seq])
